# Initial kernel scaffold; baseline (speedup 1.0000x reference)
import functools

import jax
import jax.numpy as jnp
from jax import lax
from jax.experimental import pallas as pl
from jax.experimental.pallas import tpu as pltpu
from jax.experimental.pallas import tpu_sc as plsc

N = 10000
E = 320000
H = 128
G = 128
C = 10

NC = 2
NS = 16
NPAD = 10240
ROWS_PT = NPAD // NS
EPT = E // (NC * NS)
CHUNK = 128
NCHUNK = EPT // CHUNK
TAIL = EPT - NCHUNK * CHUNK

_mesh = plsc.VectorSubcoreMesh(core_axis_name="c", subcore_axis_name="s")


@functools.partial(
    pl.kernel,
    mesh=_mesh,
    out_type=jax.ShapeDtypeStruct((NC, NPAD), jnp.float32),
    scratch_types=[
        pltpu.VMEM((CHUNK,), jnp.int32),
        pltpu.VMEM((TAIL,), jnp.int32),
        pltpu.VMEM((CHUNK,), jnp.float32),
        pltpu.VMEM((TAIL,), jnp.float32),
        pltpu.VMEM((ROWS_PT,), jnp.float32),
        pltpu.VMEM_SHARED((NPAD,), jnp.float32),
    ],
)
def _sc_degree(dst_hbm, out_hbm, idx_v, idxt_v, ones_v, onest_v, zero_v, acc):
    c = lax.axis_index("c")
    s = lax.axis_index("s")

    for i in range(CHUNK // 16):
        ones_v[pl.ds(i * 16, 16)] = jnp.ones((16,), jnp.float32)
    onest_v[pl.ds(0, 16)] = jnp.ones((16,), jnp.float32)

    def zb(i, carry):
        zero_v[pl.ds(i * 16, 16)] = jnp.zeros((16,), jnp.float32)
        return carry

    lax.fori_loop(0, ROWS_PT // 16, zb, 0)
    pltpu.sync_copy(zero_v, acc.at[pl.ds(s * ROWS_PT, ROWS_PT)])
    plsc.subcore_barrier()

    ebase = (c * NS + s) * EPT

    def body(j, carry):
        base = ebase + j * CHUNK
        pltpu.sync_copy(dst_hbm.at[pl.ds(base, CHUNK)], idx_v)
        pltpu.sync_copy(ones_v, acc.at[idx_v], add=True)
        return carry

    lax.fori_loop(0, NCHUNK, body, 0)
    pltpu.sync_copy(dst_hbm.at[pl.ds(ebase + NCHUNK * CHUNK, TAIL)], idxt_v)
    pltpu.sync_copy(onest_v, acc.at[idxt_v], add=True)

    plsc.subcore_barrier()
    pltpu.sync_copy(acc.at[pl.ds(s * ROWS_PT, ROWS_PT)],
                    out_hbm.at[c, pl.ds(s * ROWS_PT, ROWS_PT)])


@functools.partial(
    pl.kernel,
    mesh=_mesh,
    out_type=jax.ShapeDtypeStruct((NC, NPAD, H), jnp.float32),
    scratch_types=[
        pltpu.VMEM((CHUNK,), jnp.int32),
        pltpu.VMEM((CHUNK,), jnp.int32),
        pltpu.VMEM((TAIL,), jnp.int32),
        pltpu.VMEM((TAIL,), jnp.int32),
        pltpu.VMEM((CHUNK, H), jnp.float32),
        pltpu.VMEM((TAIL, H), jnp.float32),
        pltpu.VMEM((64, H), jnp.float32),
        pltpu.VMEM_SHARED((NPAD, H), jnp.float32),
        pltpu.SemaphoreType.DMA,
    ],
)
def _sc_agg(hs_hbm, src_hbm, dst_hbm, out_hbm,
            isrc, idst, isrct, idstt, rows, rowst, zer, acc, sem):
    c = lax.axis_index("c")
    s = lax.axis_index("s")

    def zb(i, carry):
        for k in range(H // 16):
            zer[i, pl.ds(k * 16, 16)] = jnp.zeros((16,), jnp.float32)
        return carry

    lax.fori_loop(0, 64, zb, 0)

    def zc(i, carry):
        pltpu.sync_copy(zer, acc.at[pl.ds(s * ROWS_PT + i * 64, 64)])
        return carry

    lax.fori_loop(0, ROWS_PT // 64, zc, 0)
    plsc.subcore_barrier()

    ebase = (c * NS + s) * EPT

    def body(j, carry):
        base = ebase + j * CHUNK
        pltpu.sync_copy(src_hbm.at[pl.ds(base, CHUNK)], isrc)
        pltpu.sync_copy(dst_hbm.at[pl.ds(base, CHUNK)], idst)
        pltpu.async_copy(hs_hbm.at[isrc], rows, sem).wait()
        pltpu.sync_copy(rows, acc.at[idst], add=True)
        return carry

    lax.fori_loop(0, NCHUNK, body, 0)

    tbase = ebase + NCHUNK * CHUNK
    pltpu.sync_copy(src_hbm.at[pl.ds(tbase, TAIL)], isrct)
    pltpu.sync_copy(dst_hbm.at[pl.ds(tbase, TAIL)], idstt)
    pltpu.async_copy(hs_hbm.at[isrct], rowst, sem).wait()
    pltpu.sync_copy(rowst, acc.at[idstt], add=True)

    plsc.subcore_barrier()
    pltpu.sync_copy(acc.at[pl.ds(s * ROWS_PT, ROWS_PT)],
                    out_hbm.at[c, pl.ds(s * ROWS_PT, ROWS_PT)])


_BLK = 2000
_GRID = N // _BLK


def _dinv_block(degp):
    deg = degp[0, :] + degp[1, :] + 1.0
    return lax.rsqrt(deg)


def _tc_in_body(x_ref, w_ref, degp_ref, h_ref, hs_ref):
    dinv = _dinv_block(degp_ref[...])
    h = jnp.dot(x_ref[...], w_ref[...], preferred_element_type=jnp.float32)
    h_ref[...] = h
    hs_ref[...] = h * dinv[:, None]


def _tc_in(x, W1, degp):
    return pl.pallas_call(
        _tc_in_body,
        grid=(_GRID,),
        in_specs=[
            pl.BlockSpec((_BLK, H), lambda i: (i, 0)),
            pl.BlockSpec((H, H), lambda i: (0, 0)),
            pl.BlockSpec((NC, _BLK), lambda i: (0, i)),
        ],
        out_specs=[
            pl.BlockSpec((_BLK, H), lambda i: (i, 0)),
            pl.BlockSpec((_BLK, H), lambda i: (i, 0)),
        ],
        out_shape=[
            jax.ShapeDtypeStruct((N, H), jnp.float32),
            jax.ShapeDtypeStruct((N, H), jnp.float32),
        ],
    )(x, W1, degp)


def _tc_mid_body(aggp_ref, h1_ref, degp_ref, b1_ref, w2_ref, h2_ref, hs2_ref):
    dinv = _dinv_block(degp_ref[...])
    agg = aggp_ref[0, :, :] + aggp_ref[1, :, :]
    h1 = h1_ref[...]
    z = jnp.maximum(
        dinv[:, None] * agg + (dinv * dinv)[:, None] * h1 + b1_ref[...][None, :],
        0.0)
    h2 = jnp.dot(z, w2_ref[...], preferred_element_type=jnp.float32)
    h2_ref[...] = h2
    hs2_ref[...] = h2 * dinv[:, None]


def _tc_mid(aggp, h1, degp, b1, W2):
    return pl.pallas_call(
        _tc_mid_body,
        grid=(_GRID,),
        in_specs=[
            pl.BlockSpec((NC, _BLK, H), lambda i: (0, i, 0)),
            pl.BlockSpec((_BLK, H), lambda i: (i, 0)),
            pl.BlockSpec((NC, _BLK), lambda i: (0, i)),
            pl.BlockSpec((H,), lambda i: (0,)),
            pl.BlockSpec((H, H), lambda i: (0, 0)),
        ],
        out_specs=[
            pl.BlockSpec((_BLK, H), lambda i: (i, 0)),
            pl.BlockSpec((_BLK, H), lambda i: (i, 0)),
        ],
        out_shape=[
            jax.ShapeDtypeStruct((N, H), jnp.float32),
            jax.ShapeDtypeStruct((N, H), jnp.float32),
        ],
    )(aggp, h1, degp, b1, W2)


def _tc_out_body(aggp_ref, h2_ref, degp_ref, b2_ref, batch_ref, wc_ref, bc_ref,
                 out_ref, pooled_acc, counts_acc):
    i = pl.program_id(0)
    dinv = _dinv_block(degp_ref[...])
    agg = aggp_ref[0, :, :] + aggp_ref[1, :, :]
    h2 = h2_ref[...]
    z = jnp.maximum(
        dinv[:, None] * agg + (dinv * dinv)[:, None] * h2 + b2_ref[...][None, :],
        0.0)
    gids = lax.broadcasted_iota(jnp.int32, (G, _BLK), 0)
    oh = (gids == batch_ref[...][None, :]).astype(jnp.float32)

    @pl.when(i == 0)
    def _():
        pooled_acc[...] = jnp.zeros((G, H), jnp.float32)
        counts_acc[...] = jnp.zeros((G, 1), jnp.float32)

    pooled_acc[...] += jnp.dot(oh, z, preferred_element_type=jnp.float32)
    counts_acc[...] += jnp.dot(oh, jnp.ones((_BLK, 1), jnp.float32),
                               preferred_element_type=jnp.float32)

    @pl.when(i == _GRID - 1)
    def _():
        pooled = pooled_acc[...] / jnp.maximum(counts_acc[...], 1.0)
        out_ref[...] = (jnp.dot(pooled, wc_ref[...],
                                preferred_element_type=jnp.float32)
                        + bc_ref[...][None, :])


def _tc_out(aggp, h2, degp, b2, batch, Wc, bc):
    return pl.pallas_call(
        _tc_out_body,
        grid=(_GRID,),
        in_specs=[
            pl.BlockSpec((NC, _BLK, H), lambda i: (0, i, 0)),
            pl.BlockSpec((_BLK, H), lambda i: (i, 0)),
            pl.BlockSpec((NC, _BLK), lambda i: (0, i)),
            pl.BlockSpec((H,), lambda i: (0,)),
            pl.BlockSpec((_BLK,), lambda i: (i,)),
            pl.BlockSpec((H, C), lambda i: (0, 0)),
            pl.BlockSpec((C,), lambda i: (0,)),
        ],
        out_specs=pl.BlockSpec((G, C), lambda i: (0, 0)),
        out_shape=jax.ShapeDtypeStruct((G, C), jnp.float32),
        scratch_shapes=[
            pltpu.VMEM((G, H), jnp.float32),
            pltpu.VMEM((G, 1), jnp.float32),
        ],
    )(aggp, h2, degp, b2, batch, Wc, bc)


def kernel(x, edge_index, batch, W1, b1, W2, b2, Wc, bc):
    src = edge_index[0]
    dst = edge_index[1]
    degp = _sc_degree(dst)
    h1, hs1 = _tc_in(x, W1, degp)
    aggp1 = _sc_agg(hs1, src, dst)
    h2, hs2 = _tc_mid(aggp1, h1, degp, b1, W2)
    aggp2 = _sc_agg(hs2, src, dst)
    return _tc_out(aggp2, h2, degp, b2, batch, Wc, bc)


# speedup vs baseline: 16.6815x; 16.6815x over previous
import functools

import jax
import jax.numpy as jnp
from jax import lax
from jax.experimental import pallas as pl
from jax.experimental.pallas import tpu as pltpu
from jax.experimental.pallas import tpu_sc as plsc

N = 10000
E = 320000
H = 128
G = 128
C = 10

NC = 2
NS = 16
NPAD = 10240
ROWS_PT = NPAD // NS
EPT = E // (NC * NS)
CHUNK = 128
NCHUNK = EPT // CHUNK
TAIL = EPT - NCHUNK * CHUNK


@functools.cache
def _sc_kernels():
    mesh = plsc.VectorSubcoreMesh(core_axis_name="c", subcore_axis_name="s")

    @functools.partial(
        pl.kernel,
        mesh=mesh,
        out_type=jax.ShapeDtypeStruct((NC, NPAD), jnp.float32),
        scratch_types=[
            pltpu.VMEM((CHUNK,), jnp.int32),
            pltpu.VMEM((TAIL,), jnp.int32),
            pltpu.VMEM((CHUNK,), jnp.float32),
            pltpu.VMEM((TAIL,), jnp.float32),
            pltpu.VMEM((ROWS_PT,), jnp.float32),
            pltpu.VMEM_SHARED((NPAD,), jnp.float32),
        ],
    )
    def sc_degree(dst_hbm, out_hbm, idx_v, idxt_v, ones_v, onest_v, zero_v,
                  acc):
        c = lax.axis_index("c")
        s = lax.axis_index("s")

        for i in range(CHUNK // 16):
            ones_v[pl.ds(i * 16, 16)] = jnp.ones((16,), jnp.float32)
        onest_v[pl.ds(0, 16)] = jnp.ones((16,), jnp.float32)

        def zb(i, carry):
            zero_v[pl.ds(i * 16, 16)] = jnp.zeros((16,), jnp.float32)
            return carry

        lax.fori_loop(0, ROWS_PT // 16, zb, 0)
        pltpu.sync_copy(zero_v, acc.at[pl.ds(s * ROWS_PT, ROWS_PT)])
        plsc.subcore_barrier()

        ebase = (c * NS + s) * EPT

        def body(j, carry):
            base = ebase + j * CHUNK
            pltpu.sync_copy(dst_hbm.at[pl.ds(base, CHUNK)], idx_v)
            pltpu.sync_copy(ones_v, acc.at[idx_v], add=True)
            return carry

        lax.fori_loop(0, NCHUNK, body, 0)
        pltpu.sync_copy(dst_hbm.at[pl.ds(ebase + NCHUNK * CHUNK, TAIL)],
                        idxt_v)
        pltpu.sync_copy(onest_v, acc.at[idxt_v], add=True)

        plsc.subcore_barrier()
        pltpu.sync_copy(acc.at[pl.ds(s * ROWS_PT, ROWS_PT)],
                        out_hbm.at[c, pl.ds(s * ROWS_PT, ROWS_PT)])

    @functools.partial(
        pl.kernel,
        mesh=mesh,
        out_type=jax.ShapeDtypeStruct((NC, NPAD, H), jnp.float32),
        scratch_types=[
            pltpu.VMEM((CHUNK,), jnp.int32),
            pltpu.VMEM((CHUNK,), jnp.int32),
            pltpu.VMEM((TAIL,), jnp.int32),
            pltpu.VMEM((TAIL,), jnp.int32),
            pltpu.VMEM((CHUNK, H), jnp.float32),
            pltpu.VMEM((TAIL, H), jnp.float32),
            pltpu.VMEM((64, H), jnp.float32),
            pltpu.VMEM_SHARED((NPAD, H), jnp.float32),
            pltpu.SemaphoreType.DMA,
        ],
    )
    def sc_agg(hs_hbm, src_hbm, dst_hbm, out_hbm,
               isrc, idst, isrct, idstt, rows, rowst, zer, acc, sem):
        c = lax.axis_index("c")
        s = lax.axis_index("s")

        def zb(i, carry):
            for k in range(H // 16):
                zer[i, pl.ds(k * 16, 16)] = jnp.zeros((16,), jnp.float32)
            return carry

        lax.fori_loop(0, 64, zb, 0)

        def zc(i, carry):
            pltpu.sync_copy(zer, acc.at[pl.ds(s * ROWS_PT + i * 64, 64)])
            return carry

        lax.fori_loop(0, ROWS_PT // 64, zc, 0)
        plsc.subcore_barrier()

        ebase = (c * NS + s) * EPT

        def body(j, carry):
            base = ebase + j * CHUNK
            pltpu.sync_copy(src_hbm.at[pl.ds(base, CHUNK)], isrc)
            pltpu.sync_copy(dst_hbm.at[pl.ds(base, CHUNK)], idst)
            pltpu.async_copy(hs_hbm.at[isrc], rows, sem).wait()
            pltpu.sync_copy(rows, acc.at[idst], add=True)
            return carry

        lax.fori_loop(0, NCHUNK, body, 0)

        tbase = ebase + NCHUNK * CHUNK
        pltpu.sync_copy(src_hbm.at[pl.ds(tbase, TAIL)], isrct)
        pltpu.sync_copy(dst_hbm.at[pl.ds(tbase, TAIL)], idstt)
        pltpu.async_copy(hs_hbm.at[isrct], rowst, sem).wait()
        pltpu.sync_copy(rowst, acc.at[idstt], add=True)

        plsc.subcore_barrier()
        pltpu.sync_copy(acc.at[pl.ds(s * ROWS_PT, ROWS_PT)],
                        out_hbm.at[c, pl.ds(s * ROWS_PT, ROWS_PT)])

    return sc_degree, sc_agg


_BLK = 2000
_GRID = N // _BLK


def _dinv_block(degp):
    deg = degp[:, 0] + degp[:, 1] + 1.0
    return lax.rsqrt(deg)


def _tc_in_body(x_ref, w_ref, degp_ref, h_ref, hs_ref):
    dinv = _dinv_block(degp_ref[...])
    h = jnp.dot(x_ref[...], w_ref[...], preferred_element_type=jnp.float32)
    h_ref[...] = h
    hs_ref[...] = h * dinv[:, None]


def _tc_in(x, W1, degp):
    return pl.pallas_call(
        _tc_in_body,
        grid=(_GRID,),
        in_specs=[
            pl.BlockSpec((_BLK, H), lambda i: (i, 0)),
            pl.BlockSpec((H, H), lambda i: (0, 0)),
            pl.BlockSpec((_BLK, NC), lambda i: (i, 0)),
        ],
        out_specs=[
            pl.BlockSpec((_BLK, H), lambda i: (i, 0)),
            pl.BlockSpec((_BLK, H), lambda i: (i, 0)),
        ],
        out_shape=[
            jax.ShapeDtypeStruct((N, H), jnp.float32),
            jax.ShapeDtypeStruct((N, H), jnp.float32),
        ],
    )(x, W1, degp)


def _tc_mid_body(aggp_ref, h1_ref, degp_ref, b1_ref, w2_ref, h2_ref, hs2_ref):
    dinv = _dinv_block(degp_ref[...])
    agg = aggp_ref[0, :, :] + aggp_ref[1, :, :]
    h1 = h1_ref[...]
    z = jnp.maximum(
        dinv[:, None] * agg + (dinv * dinv)[:, None] * h1
        + b1_ref[...][None, :], 0.0)
    h2 = jnp.dot(z, w2_ref[...], preferred_element_type=jnp.float32)
    h2_ref[...] = h2
    hs2_ref[...] = h2 * dinv[:, None]


def _tc_mid(aggp, h1, degp, b1, W2):
    return pl.pallas_call(
        _tc_mid_body,
        grid=(_GRID,),
        in_specs=[
            pl.BlockSpec((NC, _BLK, H), lambda i: (0, i, 0)),
            pl.BlockSpec((_BLK, H), lambda i: (i, 0)),
            pl.BlockSpec((_BLK, NC), lambda i: (i, 0)),
            pl.BlockSpec((H,), lambda i: (0,)),
            pl.BlockSpec((H, H), lambda i: (0, 0)),
        ],
        out_specs=[
            pl.BlockSpec((_BLK, H), lambda i: (i, 0)),
            pl.BlockSpec((_BLK, H), lambda i: (i, 0)),
        ],
        out_shape=[
            jax.ShapeDtypeStruct((N, H), jnp.float32),
            jax.ShapeDtypeStruct((N, H), jnp.float32),
        ],
    )(aggp, h1, degp, b1, W2)


def _tc_out_body(aggp_ref, h2_ref, degp_ref, b2_ref, batch_ref, wc_ref, bc_ref,
                 out_ref, pooled_acc, counts_acc):
    i = pl.program_id(0)
    dinv = _dinv_block(degp_ref[...])
    agg = aggp_ref[0, :, :] + aggp_ref[1, :, :]
    h2 = h2_ref[...]
    z = jnp.maximum(
        dinv[:, None] * agg + (dinv * dinv)[:, None] * h2
        + b2_ref[...][None, :], 0.0)
    gids = lax.broadcasted_iota(jnp.int32, (_BLK, G), 1)
    oh = (gids == batch_ref[...]).astype(jnp.float32)

    @pl.when(i == 0)
    def _():
        pooled_acc[...] = jnp.zeros((G, H), jnp.float32)
        counts_acc[...] = jnp.zeros((G, 1), jnp.float32)

    tn = (((0,), (0,)), ((), ()))
    pooled_acc[...] += lax.dot_general(oh, z, tn,
                                       preferred_element_type=jnp.float32)
    counts_acc[...] += lax.dot_general(oh, jnp.ones((_BLK, 1), jnp.float32),
                                       tn, preferred_element_type=jnp.float32)

    @pl.when(i == _GRID - 1)
    def _():
        pooled = pooled_acc[...] / jnp.maximum(counts_acc[...], 1.0)
        out_ref[...] = (jnp.dot(pooled, wc_ref[...],
                                preferred_element_type=jnp.float32)
                        + bc_ref[...][None, :])


def _tc_out(aggp, h2, degp, b2, batch, Wc, bc):
    return pl.pallas_call(
        _tc_out_body,
        grid=(_GRID,),
        in_specs=[
            pl.BlockSpec((NC, _BLK, H), lambda i: (0, i, 0)),
            pl.BlockSpec((_BLK, H), lambda i: (i, 0)),
            pl.BlockSpec((_BLK, NC), lambda i: (i, 0)),
            pl.BlockSpec((H,), lambda i: (0,)),
            pl.BlockSpec((_BLK, 1), lambda i: (i, 0)),
            pl.BlockSpec((H, C), lambda i: (0, 0)),
            pl.BlockSpec((C,), lambda i: (0,)),
        ],
        out_specs=pl.BlockSpec((G, C), lambda i: (0, 0)),
        out_shape=jax.ShapeDtypeStruct((G, C), jnp.float32),
        scratch_shapes=[
            pltpu.VMEM((G, H), jnp.float32),
            pltpu.VMEM((G, 1), jnp.float32),
        ],
    )(aggp, h2, degp, b2, batch, Wc, bc)


def kernel(x, edge_index, batch, W1, b1, W2, b2, Wc, bc):
    sc_degree, sc_agg = _sc_kernels()
    src = edge_index[0]
    dst = edge_index[1]
    degp = jnp.transpose(sc_degree(dst))
    batch2 = batch.reshape(N, 1)
    h1, hs1 = _tc_in(x, W1, degp)
    aggp1 = sc_agg(hs1, src, dst)
    h2, hs2 = _tc_mid(aggp1, h1, degp, b1, W2)
    aggp2 = sc_agg(hs2, src, dst)
    return _tc_out(aggp2, h2, degp, b2, batch2, Wc, bc)


# speedup vs baseline: 24.2335x; 1.4527x over previous
import functools

import jax
import jax.numpy as jnp
from jax import lax
from jax.experimental import pallas as pl
from jax.experimental.pallas import tpu as pltpu
from jax.experimental.pallas import tpu_sc as plsc

N = 10000
E = 320000
H = 128
G = 128
C = 10

NC = 2
NS = 16
NPAD = 10240
ROWS_PT = NPAD // NS
EPT = E // (NC * NS)
CHUNK = 128
NCHUNK = EPT // CHUNK
TAIL = EPT - NCHUNK * CHUNK


@functools.cache
def _sc_kernels():
    mesh = plsc.VectorSubcoreMesh(core_axis_name="c", subcore_axis_name="s")

    @functools.partial(
        pl.kernel,
        mesh=mesh,
        out_type=jax.ShapeDtypeStruct((NC, NPAD), jnp.float32),
        scratch_types=[
            pltpu.VMEM((CHUNK,), jnp.int32),
            pltpu.VMEM((TAIL,), jnp.int32),
            pltpu.VMEM((CHUNK,), jnp.float32),
            pltpu.VMEM((TAIL,), jnp.float32),
            pltpu.VMEM((ROWS_PT,), jnp.float32),
            pltpu.VMEM_SHARED((NPAD,), jnp.float32),
        ],
    )
    def sc_degree(dst_hbm, out_hbm, idx_v, idxt_v, ones_v, onest_v, zero_v,
                  acc):
        c = lax.axis_index("c")
        s = lax.axis_index("s")

        for i in range(CHUNK // 16):
            ones_v[pl.ds(i * 16, 16)] = jnp.ones((16,), jnp.float32)
        onest_v[pl.ds(0, 16)] = jnp.ones((16,), jnp.float32)

        def zb(i, carry):
            zero_v[pl.ds(i * 16, 16)] = jnp.zeros((16,), jnp.float32)
            return carry

        lax.fori_loop(0, ROWS_PT // 16, zb, 0)
        pltpu.sync_copy(zero_v, acc.at[pl.ds(s * ROWS_PT, ROWS_PT)])
        plsc.subcore_barrier()

        ebase = (c * NS + s) * EPT

        def body(j, carry):
            base = ebase + j * CHUNK
            pltpu.sync_copy(dst_hbm.at[pl.ds(base, CHUNK)], idx_v)
            pltpu.sync_copy(ones_v, acc.at[idx_v], add=True)
            return carry

        lax.fori_loop(0, NCHUNK, body, 0)
        pltpu.sync_copy(dst_hbm.at[pl.ds(ebase + NCHUNK * CHUNK, TAIL)],
                        idxt_v)
        pltpu.sync_copy(onest_v, acc.at[idxt_v], add=True)

        plsc.subcore_barrier()
        pltpu.sync_copy(acc.at[pl.ds(s * ROWS_PT, ROWS_PT)],
                        out_hbm.at[c, pl.ds(s * ROWS_PT, ROWS_PT)])

    NBUF = 2

    @functools.partial(
        pl.kernel,
        mesh=mesh,
        out_type=jax.ShapeDtypeStruct((NC, NPAD, H), jnp.float32),
        scratch_types=(
            [pltpu.VMEM((CHUNK,), jnp.int32) for _ in range(NBUF)]
            + [pltpu.VMEM((CHUNK,), jnp.int32) for _ in range(NBUF)]
            + [pltpu.VMEM((CHUNK, H), jnp.float32) for _ in range(NBUF)]
            + [
                pltpu.VMEM((TAIL,), jnp.int32),
                pltpu.VMEM((TAIL,), jnp.int32),
                pltpu.VMEM((TAIL, H), jnp.float32),
                pltpu.VMEM((64, H), jnp.float32),
                pltpu.VMEM_SHARED((NPAD, H), jnp.float32),
            ]
            + [pltpu.SemaphoreType.DMA for _ in range(2 * NBUF)]
        ),
    )
    def sc_agg(hs_hbm, src_hbm, dst_hbm, out_hbm, *refs):
        isrc = refs[0:NBUF]
        idst = refs[NBUF:2 * NBUF]
        rows = refs[2 * NBUF:3 * NBUF]
        isrct, idstt, rowst, zer, acc = refs[3 * NBUF:3 * NBUF + 5]
        gsem = refs[3 * NBUF + 5:3 * NBUF + 5 + NBUF]
        ssem = refs[3 * NBUF + 5 + NBUF:]

        c = lax.axis_index("c")
        s = lax.axis_index("s")

        def zb(i, carry):
            for k in range(H // 16):
                zer[i, pl.ds(k * 16, 16)] = jnp.zeros((16,), jnp.float32)
            return carry

        lax.fori_loop(0, 64, zb, 0)

        def zc(i, carry):
            pltpu.sync_copy(zer, acc.at[pl.ds(s * ROWS_PT + i * 64, 64)])
            return carry

        lax.fori_loop(0, ROWS_PT // 64, zc, 0)
        plsc.subcore_barrier()

        ebase = (c * NS + s) * EPT

        def load_idx(j, b):
            base = ebase + j * CHUNK
            pltpu.sync_copy(src_hbm.at[pl.ds(base, CHUNK)], isrc[b])
            pltpu.sync_copy(dst_hbm.at[pl.ds(base, CHUNK)], idst[b])

        for b in range(NBUF):
            load_idx(b, b)
            pltpu.async_copy(hs_hbm.at[isrc[b]], rows[b], gsem[b])

        def body(i, carry):
            j0 = i * NBUF
            for b in range(NBUF):
                pltpu.make_async_copy(hs_hbm.at[isrc[b]], rows[b],
                                      gsem[b]).wait()
                pltpu.async_copy(rows[b], acc.at[idst[b]], ssem[b],
                                 add=True)
            for b in range(NBUF):
                @pl.when(j0 + b + NBUF < NCHUNK)
                def _():
                    pltpu.make_async_copy(rows[b], acc.at[idst[b]],
                                          ssem[b]).wait()
                    load_idx(j0 + b + NBUF, b)
                    pltpu.async_copy(hs_hbm.at[isrc[b]], rows[b], gsem[b])
            return carry

        lax.fori_loop(0, NCHUNK // NBUF, body, 0)

        for b in range(NBUF):
            pltpu.make_async_copy(rows[b], acc.at[idst[b]], ssem[b]).wait()

        tbase = ebase + NCHUNK * CHUNK
        pltpu.sync_copy(src_hbm.at[pl.ds(tbase, TAIL)], isrct)
        pltpu.sync_copy(dst_hbm.at[pl.ds(tbase, TAIL)], idstt)
        pltpu.async_copy(hs_hbm.at[isrct], rowst, gsem[0]).wait()
        pltpu.sync_copy(rowst, acc.at[idstt], add=True)

        plsc.subcore_barrier()
        pltpu.sync_copy(acc.at[pl.ds(s * ROWS_PT, ROWS_PT)],
                        out_hbm.at[c, pl.ds(s * ROWS_PT, ROWS_PT)])

    return sc_degree, sc_agg


_BLK = 2000
_GRID = N // _BLK


def _dinv_block(degp):
    deg = degp[:, 0] + degp[:, 1] + 1.0
    return lax.rsqrt(deg)


def _tc_in_body(x_ref, w_ref, degp_ref, h_ref, hs_ref):
    dinv = _dinv_block(degp_ref[...])
    h = jnp.dot(x_ref[...], w_ref[...], preferred_element_type=jnp.float32)
    h_ref[...] = h
    hs_ref[...] = h * dinv[:, None]


def _tc_in(x, W1, degp):
    return pl.pallas_call(
        _tc_in_body,
        grid=(_GRID,),
        in_specs=[
            pl.BlockSpec((_BLK, H), lambda i: (i, 0)),
            pl.BlockSpec((H, H), lambda i: (0, 0)),
            pl.BlockSpec((_BLK, NC), lambda i: (i, 0)),
        ],
        out_specs=[
            pl.BlockSpec((_BLK, H), lambda i: (i, 0)),
            pl.BlockSpec((_BLK, H), lambda i: (i, 0)),
        ],
        out_shape=[
            jax.ShapeDtypeStruct((N, H), jnp.float32),
            jax.ShapeDtypeStruct((N, H), jnp.float32),
        ],
    )(x, W1, degp)


def _tc_mid_body(aggp_ref, h1_ref, degp_ref, b1_ref, w2_ref, h2_ref, hs2_ref):
    dinv = _dinv_block(degp_ref[...])
    agg = aggp_ref[0, :, :] + aggp_ref[1, :, :]
    h1 = h1_ref[...]
    z = jnp.maximum(
        dinv[:, None] * agg + (dinv * dinv)[:, None] * h1
        + b1_ref[...][None, :], 0.0)
    h2 = jnp.dot(z, w2_ref[...], preferred_element_type=jnp.float32)
    h2_ref[...] = h2
    hs2_ref[...] = h2 * dinv[:, None]


def _tc_mid(aggp, h1, degp, b1, W2):
    return pl.pallas_call(
        _tc_mid_body,
        grid=(_GRID,),
        in_specs=[
            pl.BlockSpec((NC, _BLK, H), lambda i: (0, i, 0)),
            pl.BlockSpec((_BLK, H), lambda i: (i, 0)),
            pl.BlockSpec((_BLK, NC), lambda i: (i, 0)),
            pl.BlockSpec((H,), lambda i: (0,)),
            pl.BlockSpec((H, H), lambda i: (0, 0)),
        ],
        out_specs=[
            pl.BlockSpec((_BLK, H), lambda i: (i, 0)),
            pl.BlockSpec((_BLK, H), lambda i: (i, 0)),
        ],
        out_shape=[
            jax.ShapeDtypeStruct((N, H), jnp.float32),
            jax.ShapeDtypeStruct((N, H), jnp.float32),
        ],
    )(aggp, h1, degp, b1, W2)


def _tc_out_body(aggp_ref, h2_ref, degp_ref, b2_ref, batch_ref, wc_ref, bc_ref,
                 out_ref, pooled_acc, counts_acc):
    i = pl.program_id(0)
    dinv = _dinv_block(degp_ref[...])
    agg = aggp_ref[0, :, :] + aggp_ref[1, :, :]
    h2 = h2_ref[...]
    z = jnp.maximum(
        dinv[:, None] * agg + (dinv * dinv)[:, None] * h2
        + b2_ref[...][None, :], 0.0)
    gids = lax.broadcasted_iota(jnp.int32, (_BLK, G), 1)
    oh = (gids == batch_ref[...]).astype(jnp.float32)

    @pl.when(i == 0)
    def _():
        pooled_acc[...] = jnp.zeros((G, H), jnp.float32)
        counts_acc[...] = jnp.zeros((G, 1), jnp.float32)

    tn = (((0,), (0,)), ((), ()))
    pooled_acc[...] += lax.dot_general(oh, z, tn,
                                       preferred_element_type=jnp.float32)
    counts_acc[...] += lax.dot_general(oh, jnp.ones((_BLK, 1), jnp.float32),
                                       tn, preferred_element_type=jnp.float32)

    @pl.when(i == _GRID - 1)
    def _():
        pooled = pooled_acc[...] / jnp.maximum(counts_acc[...], 1.0)
        out_ref[...] = (jnp.dot(pooled, wc_ref[...],
                                preferred_element_type=jnp.float32)
                        + bc_ref[...][None, :])


def _tc_out(aggp, h2, degp, b2, batch, Wc, bc):
    return pl.pallas_call(
        _tc_out_body,
        grid=(_GRID,),
        in_specs=[
            pl.BlockSpec((NC, _BLK, H), lambda i: (0, i, 0)),
            pl.BlockSpec((_BLK, H), lambda i: (i, 0)),
            pl.BlockSpec((_BLK, NC), lambda i: (i, 0)),
            pl.BlockSpec((H,), lambda i: (0,)),
            pl.BlockSpec((_BLK, 1), lambda i: (i, 0)),
            pl.BlockSpec((H, C), lambda i: (0, 0)),
            pl.BlockSpec((C,), lambda i: (0,)),
        ],
        out_specs=pl.BlockSpec((G, C), lambda i: (0, 0)),
        out_shape=jax.ShapeDtypeStruct((G, C), jnp.float32),
        scratch_shapes=[
            pltpu.VMEM((G, H), jnp.float32),
            pltpu.VMEM((G, 1), jnp.float32),
        ],
    )(aggp, h2, degp, b2, batch, Wc, bc)


def kernel(x, edge_index, batch, W1, b1, W2, b2, Wc, bc):
    sc_degree, sc_agg = _sc_kernels()
    src = edge_index[0]
    dst = edge_index[1]
    degp = jnp.transpose(sc_degree(dst))
    batch2 = batch.reshape(N, 1)
    h1, hs1 = _tc_in(x, W1, degp)
    aggp1 = sc_agg(hs1, src, dst)
    h2, hs2 = _tc_mid(aggp1, h1, degp, b1, W2)
    aggp2 = sc_agg(hs2, src, dst)
    return _tc_out(aggp2, h2, degp, b2, batch2, Wc, bc)
